# baseline (speedup 1.0000x reference)
import functools

import jax
import jax.numpy as jnp
from jax import lax
from jax.experimental import pallas as pl
from jax.experimental.pallas import tpu as pltpu
from jax.experimental.pallas import tpu_sc as plsc

NC = 2
NS = 16
NW = NC * NS
NBUF = 4
DEPTH = 2

V = 1000000
D = 32
L = 16
N_B = 4096
N_L = 200

_mesh = plsc.VectorSubcoreMesh(core_axis_name="c", subcore_axis_name="s")

_A_VBLK = 32768
_A_GRID = pl.cdiv(V, _A_VBLK)
_TAB_ROWS = _A_GRID * (_A_VBLK * D // 128)
_TAB_V = _TAB_ROWS * 128 // D


def _detile_body(wt_ref, out_ref):
  x = wt_ref[...]
  for g in range(_A_VBLK // 512):
    for u in range(4):
      out_ref[128 * g:128 * (g + 1), 32 * u:32 * (u + 1)] = (
          x[:, 512 * g + 128 * u:512 * g + 128 * (u + 1)].T)


_detile = pl.pallas_call(
    _detile_body,
    grid=(_A_GRID,),
    in_specs=[pl.BlockSpec((D, _A_VBLK), lambda j: (0, j))],
    out_specs=pl.BlockSpec((_A_VBLK * D // 128, 128), lambda j: (j, 0)),
    out_shape=jax.ShapeDtypeStruct((_TAB_ROWS, 128), jnp.float32),
    compiler_params=pltpu.CompilerParams(dimension_semantics=("parallel",)),
)


_NCH = N_L // 4


@functools.partial(
    pl.kernel,
    out_type=jax.ShapeDtypeStruct((N_L // 4, N_B, 128), jnp.float32),
    mesh=_mesh,
    scratch_types=[
        pltpu.VMEM((_NCH, 512), jnp.int32),
        [pltpu.VMEM((512, D), jnp.float32) for _ in range(NBUF)],
        [pltpu.SemaphoreType.DMA for _ in range(NBUF)],
        [pltpu.SemaphoreType.DMA for _ in range(NBUF)],
    ],
    compiler_params=pltpu.CompilerParams(use_tc_tiling_on_sc=False),
)
def _gather(idx_hbm, w_hbm, out_hbm, idx_v, bufs, gsems, ssems):
  wid = lax.axis_index("s") * NC + lax.axis_index("c")
  b0 = wid * 128
  pltpu.sync_copy(idx_hbm.at[:, pl.ds(wid * 512, 512)], idx_v)

  @pl.loop(0, _NCH)
  def _remap(r):
    for g in range(512 // L):
      v = idx_v[r, pl.ds(g * L, L)]
      t = (lax.bitwise_and(v, jnp.int32(~511))
           + lax.shift_left(lax.bitwise_and(v, jnp.int32(127)), 2)
           + lax.bitwise_and(lax.shift_right_logical(v, 7), jnp.int32(3)))
      idx_v[r, pl.ds(g * L, L)] = t

  def fire_gather(c, slot):
    pltpu.async_copy(w_hbm.at[idx_v.at[c]], bufs[slot], gsems[slot])

  def wait_gather(slot):
    pltpu.make_async_copy(w_hbm.at[idx_v.at[0]], bufs[slot],
                          gsems[slot]).wait()

  def fire_stores(c, slot):
    for e in range(4):
      pltpu.async_copy(
          bufs[slot].at[pl.ds(128 * e, 128)],
          out_hbm.at[c, pl.ds(b0, 128), pl.ds(pl.multiple_of(e * D, D), D)],
          ssems[slot])

  def wait_stores(slot):
    for e in range(4):
      pltpu.make_async_copy(bufs[slot].at[pl.ds(0, 128)],
                            out_hbm.at[0, pl.ds(b0, 128), pl.ds(0, D)],
                            ssems[slot]).wait()

  for c in range(DEPTH):
    fire_gather(c, c % NBUF)
  for c in range(_NCH):
    if c + DEPTH < _NCH:
      if c + DEPTH >= NBUF:
        wait_stores((c + DEPTH) % NBUF)
      fire_gather(c + DEPTH, (c + DEPTH) % NBUF)
    wait_gather(c % NBUF)
    fire_stores(c, c % NBUF)
  for c in range(_NCH - NBUF, _NCH):
    wait_stores(c % NBUF)


_C_WBLK = 32
_C_LBLK = 5


def _tile_body(g_ref, out_ref):
  x = g_ref[...]
  for t in range(_C_LBLK):
    for w in range(_C_WBLK):
      y = x[t, 128 * w:128 * (w + 1), :].T
      for e in range(4):
        out_ref[t, e, :, 128 * w:128 * (w + 1)] = y[32 * e:32 * (e + 1), :]


_tile_out = pl.pallas_call(
    _tile_body,
    grid=(N_L // 4 // _C_LBLK, N_B // (128 * _C_WBLK)),
    in_specs=[pl.BlockSpec((_C_LBLK, 128 * _C_WBLK, 128),
                           lambda l4, j: (l4, j, 0))],
    out_specs=pl.BlockSpec((_C_LBLK, 4, D, 128 * _C_WBLK),
                           lambda l4, j: (l4, 0, 0, j)),
    out_shape=jax.ShapeDtypeStruct((N_L // 4, 4, D, N_B), jnp.float32),
    compiler_params=pltpu.CompilerParams(
        dimension_semantics=("parallel", "parallel")),
)


def kernel(x, weight):
  table = _detile(weight.T).reshape(_TAB_V, D)
  idx = (x.astype(jnp.int32)
         .reshape(NW, 128, N_L // 4, 4)
         .transpose(2, 0, 3, 1)
         .reshape(N_L // 4, NW * 512))
  g = _gather(idx, table)
  oc = _tile_out(g).reshape(N_L, D, N_B)
  return oc.transpose(2, 0, 1)


# speedup vs baseline: 1.0143x; 1.0143x over previous
import functools

import jax
import jax.numpy as jnp
from jax import lax
from jax.experimental import pallas as pl
from jax.experimental.pallas import tpu as pltpu
from jax.experimental.pallas import tpu_sc as plsc

NC = 2
NS = 16
NW = NC * NS
NBUF = 8
DEPTH = 4

V = 1000000
D = 32
L = 16
N_B = 4096
N_L = 200

_mesh = plsc.VectorSubcoreMesh(core_axis_name="c", subcore_axis_name="s")

_A_VBLK = 32768
_A_GRID = pl.cdiv(V, _A_VBLK)
_TAB_ROWS = _A_GRID * (_A_VBLK * D // 128)
_TAB_V = _TAB_ROWS * 128 // D


def _detile_body(wt_ref, out_ref):
  x = wt_ref[...]
  for g in range(_A_VBLK // 512):
    for u in range(4):
      out_ref[128 * g:128 * (g + 1), 32 * u:32 * (u + 1)] = (
          x[:, 512 * g + 128 * u:512 * g + 128 * (u + 1)].T)


_detile = pl.pallas_call(
    _detile_body,
    grid=(_A_GRID,),
    in_specs=[pl.BlockSpec((D, _A_VBLK), lambda j: (0, j))],
    out_specs=pl.BlockSpec((_A_VBLK * D // 128, 128), lambda j: (j, 0)),
    out_shape=jax.ShapeDtypeStruct((_TAB_ROWS, 128), jnp.float32),
    compiler_params=pltpu.CompilerParams(dimension_semantics=("parallel",)),
)


@functools.partial(
    pl.kernel,
    out_type=jax.ShapeDtypeStruct((N_L // 4, N_B, 128), jnp.float32),
    mesh=_mesh,
    scratch_types=[
        pltpu.VMEM((N_L, 128), jnp.int32),
        [pltpu.VMEM((128, D), jnp.float32) for _ in range(NBUF)],
        [pltpu.SemaphoreType.DMA for _ in range(NBUF)],
        [pltpu.SemaphoreType.DMA for _ in range(NBUF)],
    ],
    compiler_params=pltpu.CompilerParams(use_tc_tiling_on_sc=False),
)
def _gather(idx_hbm, w_hbm, out_hbm, idx_v, bufs, gsems, ssems):
  wid = lax.axis_index("s") * NC + lax.axis_index("c")
  b0 = wid * 128
  pltpu.sync_copy(idx_hbm.at[:, pl.ds(b0, 128)], idx_v)

  @pl.loop(0, N_L)
  def _remap(r):
    for g in range(128 // L):
      v = idx_v[r, pl.ds(g * L, L)]
      t = (lax.bitwise_and(v, jnp.int32(~511))
           + lax.shift_left(lax.bitwise_and(v, jnp.int32(127)), 2)
           + lax.bitwise_and(lax.shift_right_logical(v, 7), jnp.int32(3)))
      idx_v[r, pl.ds(g * L, L)] = t

  def fire_gather(c, slot):
    pltpu.async_copy(w_hbm.at[idx_v.at[c]], bufs[slot], gsems[slot])

  def wait_gather(slot):
    pltpu.make_async_copy(w_hbm.at[idx_v.at[0]], bufs[slot],
                          gsems[slot]).wait()

  def fire_store(c, slot):
    if isinstance(c, int):
      l4, e = c // 4, c % 4
    else:
      l4, e = lax.div(c, 4), lax.rem(c, 4)
    pltpu.async_copy(
        bufs[slot],
        out_hbm.at[l4, pl.ds(b0, 128), pl.ds(pl.multiple_of(e * D, D), D)],
        ssems[slot])

  def wait_store(slot):
    pltpu.make_async_copy(bufs[slot],
                          out_hbm.at[0, pl.ds(b0, 128), pl.ds(0, D)],
                          ssems[slot]).wait()

  for c in range(DEPTH):
    fire_gather(c, c % NBUF)
  for c in range(DEPTH, NBUF):
    fire_gather(c, c % NBUF)
    wait_gather((c - DEPTH) % NBUF)
    fire_store(c - DEPTH, (c - DEPTH) % NBUF)

  @pl.loop(NBUF, N_L, step=NBUF)
  def _body(j):
    for b in range(NBUF):
      c = j + b
      wait_store(b)
      fire_gather(c, b)
      bd = (b - DEPTH) % NBUF
      wait_gather(bd)
      fire_store(c - DEPTH, bd)

  for c in range(N_L, N_L + DEPTH):
    b = c % NBUF
    wait_store(b)
    bd = (c - DEPTH) % NBUF
    wait_gather(bd)
    fire_store(c - DEPTH, bd)
  for c in range(N_L + DEPTH, N_L + NBUF):
    wait_store(c % NBUF)


_C_WBLK = 32
_C_LBLK = 5


def _tile_body(g_ref, out_ref):
  x = g_ref[...]
  for t in range(_C_LBLK):
    for w in range(_C_WBLK):
      y = x[t, 128 * w:128 * (w + 1), :].T
      for e in range(4):
        out_ref[t, e, :, 128 * w:128 * (w + 1)] = y[32 * e:32 * (e + 1), :]


_tile_out = pl.pallas_call(
    _tile_body,
    grid=(N_L // 4 // _C_LBLK, N_B // (128 * _C_WBLK)),
    in_specs=[pl.BlockSpec((_C_LBLK, 128 * _C_WBLK, 128),
                           lambda l4, j: (l4, j, 0))],
    out_specs=pl.BlockSpec((_C_LBLK, 4, D, 128 * _C_WBLK),
                           lambda l4, j: (l4, 0, 0, j)),
    out_shape=jax.ShapeDtypeStruct((N_L // 4, 4, D, N_B), jnp.float32),
    compiler_params=pltpu.CompilerParams(
        dimension_semantics=("parallel", "parallel")),
)


def kernel(x, weight):
  table = _detile(weight.T).reshape(_TAB_V, D)
  g = _gather(x.T.astype(jnp.int32), table)
  oc = _tile_out(g).reshape(N_L, D, N_B)
  return oc.transpose(2, 0, 1)


# speedup vs baseline: 1.6547x; 1.6314x over previous
import functools

import jax
import jax.numpy as jnp
from jax import lax
from jax.experimental import pallas as pl
from jax.experimental.pallas import tpu as pltpu
from jax.experimental.pallas import tpu_sc as plsc

NC = 2
NS = 16
NW = NC * NS
NBUF = 8
DEPTH = 4

V = 1000000
D = 32
L = 16
N_B = 4096
N_L = 200

_mesh = plsc.VectorSubcoreMesh(core_axis_name="c", subcore_axis_name="s")

_A_VBLK = 32768
_A_GRID = pl.cdiv(V, _A_VBLK)
_TAB_ROWS = _A_GRID * (_A_VBLK * D // 128)
_TAB_V = _TAB_ROWS * 128 // D


def _detile_body(wt_ref, out_ref):
  x = wt_ref[...]
  for g in range(_A_VBLK // 512):
    y = jnp.concatenate(
        [x[:, 512 * g + 128 * u:512 * g + 128 * (u + 1)] for u in range(4)],
        axis=0)
    out_ref[128 * g:128 * (g + 1), :] = y.T


_detile = pl.pallas_call(
    _detile_body,
    grid=(_A_GRID,),
    in_specs=[pl.BlockSpec((D, _A_VBLK), lambda j: (0, j))],
    out_specs=pl.BlockSpec((_A_VBLK * D // 128, 128), lambda j: (j, 0)),
    out_shape=jax.ShapeDtypeStruct((_TAB_ROWS, 128), jnp.float32),
    compiler_params=pltpu.CompilerParams(dimension_semantics=("parallel",)),
)


@functools.partial(
    pl.kernel,
    out_type=jax.ShapeDtypeStruct((N_L // 4, N_B, 128), jnp.float32),
    mesh=_mesh,
    scratch_types=[
        pltpu.VMEM((N_L, 128), jnp.int32),
        [pltpu.VMEM((128, D), jnp.float32) for _ in range(NBUF)],
        [pltpu.SemaphoreType.DMA for _ in range(NBUF)],
        [pltpu.SemaphoreType.DMA for _ in range(NBUF)],
    ],
    compiler_params=pltpu.CompilerParams(use_tc_tiling_on_sc=False),
)
def _gather(idx_hbm, w_hbm, out_hbm, idx_v, bufs, gsems, ssems):
  wid = lax.axis_index("s") * NC + lax.axis_index("c")
  b0 = wid * 128
  pltpu.sync_copy(idx_hbm.at[:, pl.ds(b0, 128)], idx_v)

  @pl.loop(0, N_L)
  def _remap(r):
    for g in range(128 // L):
      v = idx_v[r, pl.ds(g * L, L)]
      t = (lax.bitwise_and(v, jnp.int32(~511))
           + lax.shift_left(lax.bitwise_and(v, jnp.int32(127)), 2)
           + lax.bitwise_and(lax.shift_right_logical(v, 7), jnp.int32(3)))
      idx_v[r, pl.ds(g * L, L)] = t

  def fire_gather(c, slot):
    pltpu.async_copy(w_hbm.at[idx_v.at[c]], bufs[slot], gsems[slot])

  def wait_gather(slot):
    pltpu.make_async_copy(w_hbm.at[idx_v.at[0]], bufs[slot],
                          gsems[slot]).wait()

  def fire_store(c, slot):
    if isinstance(c, int):
      l4, e = c // 4, c % 4
    else:
      l4, e = lax.div(c, 4), lax.rem(c, 4)
    pltpu.async_copy(
        bufs[slot],
        out_hbm.at[l4, pl.ds(b0, 128), pl.ds(pl.multiple_of(e * D, D), D)],
        ssems[slot])

  def wait_store(slot):
    pltpu.make_async_copy(bufs[slot],
                          out_hbm.at[0, pl.ds(b0, 128), pl.ds(0, D)],
                          ssems[slot]).wait()

  for c in range(DEPTH):
    fire_gather(c, c % NBUF)
  for c in range(DEPTH, NBUF):
    fire_gather(c, c % NBUF)
    wait_gather((c - DEPTH) % NBUF)
    fire_store(c - DEPTH, (c - DEPTH) % NBUF)

  @pl.loop(NBUF, N_L, step=NBUF)
  def _body(j):
    for b in range(NBUF):
      c = j + b
      wait_store(b)
      fire_gather(c, b)
      bd = (b - DEPTH) % NBUF
      wait_gather(bd)
      fire_store(c - DEPTH, bd)

  for c in range(N_L, N_L + DEPTH):
    b = c % NBUF
    wait_store(b)
    bd = (c - DEPTH) % NBUF
    wait_gather(bd)
    fire_store(c - DEPTH, bd)
  for c in range(N_L + DEPTH, N_L + NBUF):
    wait_store(c % NBUF)


_C_WBLK = 32
_C_LBLK = 5


def _tile_body(g_ref, out_ref):
  x = g_ref[...]
  for t in range(_C_LBLK):
    for w in range(_C_WBLK):
      y = x[t, 128 * w:128 * (w + 1), :].T
      for e in range(4):
        out_ref[t, e, :, 128 * w:128 * (w + 1)] = y[32 * e:32 * (e + 1), :]


_tile_out = pl.pallas_call(
    _tile_body,
    grid=(N_L // 4 // _C_LBLK, N_B // (128 * _C_WBLK)),
    in_specs=[pl.BlockSpec((_C_LBLK, 128 * _C_WBLK, 128),
                           lambda l4, j: (l4, j, 0))],
    out_specs=pl.BlockSpec((_C_LBLK, 4, D, 128 * _C_WBLK),
                           lambda l4, j: (l4, 0, 0, j)),
    out_shape=jax.ShapeDtypeStruct((N_L // 4, 4, D, N_B), jnp.float32),
    compiler_params=pltpu.CompilerParams(
        dimension_semantics=("parallel", "parallel")),
)


def kernel(x, weight):
  table = _detile(weight.T).reshape(_TAB_V, D)
  g = _gather(x.T.astype(jnp.int32), table)
  oc = _tile_out(g).reshape(N_L, D, N_B)
  return oc.transpose(2, 0, 1)


# speedup vs baseline: 1.6548x; 1.0000x over previous
import functools

import jax
import jax.numpy as jnp
from jax import lax
from jax.experimental import pallas as pl
from jax.experimental.pallas import tpu as pltpu
from jax.experimental.pallas import tpu_sc as plsc

NC = 2
NS = 16
NW = NC * NS
NBUF = 8
DEPTH = 4

V = 1000000
D = 32
L = 16
N_B = 4096
N_L = 200

_mesh = plsc.VectorSubcoreMesh(core_axis_name="c", subcore_axis_name="s")

_A_VBLK = 32768
_A_GRID = pl.cdiv(V, _A_VBLK)
_TAB_ROWS = _A_GRID * (_A_VBLK * D // 128)
_TAB_V = _TAB_ROWS * 128 // D


def _detile_body(wt_ref, out_ref):
  x = wt_ref[...]
  for g in range(_A_VBLK // 512):
    y = jnp.concatenate(
        [x[:, 512 * g + 128 * u:512 * g + 128 * (u + 1)] for u in range(4)],
        axis=0)
    out_ref[128 * g:128 * (g + 1), :] = y.T


_detile = pl.pallas_call(
    _detile_body,
    grid=(_A_GRID,),
    in_specs=[pl.BlockSpec((D, _A_VBLK), lambda j: (0, j))],
    out_specs=pl.BlockSpec((_A_VBLK * D // 128, 128), lambda j: (j, 0)),
    out_shape=jax.ShapeDtypeStruct((_TAB_ROWS, 128), jnp.float32),
    compiler_params=pltpu.CompilerParams(dimension_semantics=("parallel",)),
)


@functools.partial(
    pl.kernel,
    out_type=jax.ShapeDtypeStruct((N_L // 4, N_B, 128), jnp.float32),
    mesh=_mesh,
    scratch_types=[
        pltpu.VMEM((N_L, 128), jnp.int32),
        [pltpu.VMEM((128, D), jnp.float32) for _ in range(NBUF)],
        [pltpu.SemaphoreType.DMA for _ in range(NBUF)],
        [pltpu.SemaphoreType.DMA for _ in range(NBUF)],
    ],
    compiler_params=pltpu.CompilerParams(use_tc_tiling_on_sc=False),
)
def _gather(idx_hbm, w_hbm, out_hbm, idx_v, bufs, gsems, ssems):
  wid = lax.axis_index("s") * NC + lax.axis_index("c")
  b0 = wid * 128
  pltpu.sync_copy(idx_hbm.at[:, pl.ds(b0, 128)], idx_v)

  @pl.loop(0, N_L)
  def _remap(r):
    for g in range(128 // L):
      v = idx_v[r, pl.ds(g * L, L)]
      t = (lax.bitwise_and(v, jnp.int32(~511))
           + lax.shift_left(lax.bitwise_and(v, jnp.int32(127)), 2)
           + lax.bitwise_and(lax.shift_right_logical(v, 7), jnp.int32(3)))
      idx_v[r, pl.ds(g * L, L)] = t

  def fire_gather(c, slot):
    pltpu.async_copy(w_hbm.at[idx_v.at[c]], bufs[slot], gsems[slot])

  def wait_gather(slot):
    pltpu.make_async_copy(w_hbm.at[idx_v.at[0]], bufs[slot],
                          gsems[slot]).wait()

  def fire_store(c, slot):
    if isinstance(c, int):
      l4, e = c // 4, c % 4
    else:
      l4, e = lax.div(c, 4), lax.rem(c, 4)
    pltpu.async_copy(
        bufs[slot],
        out_hbm.at[l4, pl.ds(b0, 128), pl.ds(pl.multiple_of(e * D, D), D)],
        ssems[slot])

  def wait_store(slot):
    pltpu.make_async_copy(bufs[slot],
                          out_hbm.at[0, pl.ds(b0, 128), pl.ds(0, D)],
                          ssems[slot]).wait()

  for c in range(DEPTH):
    fire_gather(c, c % NBUF)
  for c in range(DEPTH, NBUF):
    fire_gather(c, c % NBUF)
    wait_gather((c - DEPTH) % NBUF)
    fire_store(c - DEPTH, (c - DEPTH) % NBUF)

  @pl.loop(NBUF, N_L, step=NBUF)
  def _body(j):
    for b in range(NBUF):
      c = j + b
      wait_store(b)
      fire_gather(c, b)
      bd = (b - DEPTH) % NBUF
      wait_gather(bd)
      fire_store(c - DEPTH, bd)

  for c in range(N_L, N_L + DEPTH):
    b = c % NBUF
    wait_store(b)
    bd = (c - DEPTH) % NBUF
    wait_gather(bd)
    fire_store(c - DEPTH, bd)
  for c in range(N_L + DEPTH, N_L + NBUF):
    wait_store(c % NBUF)


_C_WBLK = 32
_C_LBLK = 5


def _tile_body(g_ref, out_ref):
  x = g_ref[...]
  for t in range(_C_LBLK):
    for w in range(_C_WBLK):
      out_ref[t, :, 128 * w:128 * (w + 1)] = (
          x[t, 128 * w:128 * (w + 1), :].T)


_tile_out = pl.pallas_call(
    _tile_body,
    grid=(N_L // 4 // _C_LBLK, N_B // (128 * _C_WBLK)),
    in_specs=[pl.BlockSpec((_C_LBLK, 128 * _C_WBLK, 128),
                           lambda l4, j: (l4, j, 0))],
    out_specs=pl.BlockSpec((_C_LBLK, 4 * D, 128 * _C_WBLK),
                           lambda l4, j: (l4, 0, j)),
    out_shape=jax.ShapeDtypeStruct((N_L // 4, 4 * D, N_B), jnp.float32),
    compiler_params=pltpu.CompilerParams(
        dimension_semantics=("parallel", "parallel")),
)


def kernel(x, weight):
  table = _detile(weight.T).reshape(_TAB_V, D)
  g = _gather(x.T.astype(jnp.int32), table)
  oc = _tile_out(g).reshape(N_L, D, N_B)
  return oc.transpose(2, 0, 1)
